# baseline (speedup 1.0000x reference)
import functools

import jax
import jax.numpy as jnp
from jax import lax
from jax.experimental import pallas as pl
from jax.experimental.pallas import tpu as pltpu
from jax.experimental.pallas import tpu_sc as plsc

_B = 4
_S = 4096
_D = 1024
_LANES = 16
_NC = 2
_NS = 16
_NW = _NC * _NS
_N = _B * _S
_RPW = _N // _NW
_CH = 32
_NCH = _RPW // _CH


def _make_kernel():
    mesh = plsc.VectorSubcoreMesh(core_axis_name="c", subcore_axis_name="s")

    @functools.partial(
        pl.kernel,
        out_type=jax.ShapeDtypeStruct((_N, _D), jnp.float32),
        mesh=mesh,
        scratch_types=[
            pltpu.VMEM((_NCH, _CH), jnp.int32),
            pltpu.VMEM((_CH, _D), jnp.float32),
            pltpu.VMEM((_CH, _D), jnp.float32),
            pltpu.SemaphoreType.DMA,
        ],
    )
    def body(ids_hbm, tok_hbm, pos_hbm, out_hbm, idx_v, tokbuf, posbuf, sem):
        wid = lax.axis_index("s") * _NC + lax.axis_index("c")
        base = wid * _RPW
        pos_base = lax.rem(base, _S)
        pltpu.sync_copy(ids_hbm.at[wid], idx_v)

        def chunk(i, carry):
            pltpu.async_copy(tok_hbm.at[idx_v.at[i]], tokbuf, sem).wait()
            pltpu.sync_copy(pos_hbm.at[pl.ds(pos_base + i * _CH, _CH)], posbuf)

            def add_row(r, c2):
                for c in range(_D // _LANES):
                    sl = pl.ds(c * _LANES, _LANES)
                    tokbuf[r, sl] = tokbuf[r, sl] + posbuf[r, sl]
                return c2

            lax.fori_loop(0, _CH, add_row, 0)
            pltpu.sync_copy(tokbuf, out_hbm.at[pl.ds(base + i * _CH, _CH)])
            return carry

        lax.fori_loop(0, _NCH, chunk, 0)

    return body


_kernel_fn = _make_kernel()


def kernel(input_ids, token_table, pos_table):
    ids = input_ids.astype(jnp.int32).reshape(_NW, _NCH, _CH)
    out = _kernel_fn(ids, token_table, pos_table)
    return out.reshape(_B, _S, _D)


# speedup vs baseline: 1.3804x; 1.3804x over previous
import functools

import jax
import jax.numpy as jnp
from jax import lax
from jax.experimental import pallas as pl
from jax.experimental.pallas import tpu as pltpu
from jax.experimental.pallas import tpu_sc as plsc

_B = 4
_S = 4096
_D = 1024
_LANES = 16
_NC = 2
_NS = 16
_NW = _NC * _NS
_N = _B * _S
_RPW = _N // _NW
_CH = 16
_NCH = _RPW // _CH
_NBUF = 2


def _make_kernel():
    mesh = plsc.VectorSubcoreMesh(core_axis_name="c", subcore_axis_name="s")

    @functools.partial(
        pl.kernel,
        out_type=jax.ShapeDtypeStruct((_N, _D), jnp.float32),
        mesh=mesh,
        scratch_types=[
            pltpu.VMEM((_NCH, _CH), jnp.int32),
            pltpu.VMEM((_NBUF, _CH, _D), jnp.float32),
            pltpu.VMEM((_NBUF, _CH, _D), jnp.float32),
            pltpu.VMEM((_NBUF, _CH, _D), jnp.float32),
        ] + [pltpu.SemaphoreType.DMA] * (3 * _NBUF),
    )
    def body(ids_hbm, tok_hbm, pos_hbm, out_hbm, idx_v, tkb, psb, ob,
             g0, g1, p0, p1, o0, o1):
        gs = (g0, g1)
        ps = (p0, p1)
        osm = (o0, o1)
        wid = lax.axis_index("s") * _NC + lax.axis_index("c")
        base = wid * _RPW
        pos_base = lax.rem(base, _S)
        pltpu.sync_copy(ids_hbm.at[wid], idx_v)

        def start_g(i, b):
            pltpu.async_copy(tok_hbm.at[idx_v.at[i]], tkb.at[b], gs[b])
            pltpu.async_copy(pos_hbm.at[pl.ds(pos_base + i * _CH, _CH)],
                             psb.at[b], ps[b])

        def wait_g(b):
            pltpu.make_async_copy(tok_hbm.at[pl.ds(0, _CH)], tkb.at[b],
                                  gs[b]).wait()
            pltpu.make_async_copy(pos_hbm.at[pl.ds(0, _CH)], psb.at[b],
                                  ps[b]).wait()

        def start_o(i, b):
            pltpu.async_copy(ob.at[b], out_hbm.at[pl.ds(base + i * _CH, _CH)],
                             osm[b])

        def wait_o(b):
            pltpu.make_async_copy(ob.at[b], out_hbm.at[pl.ds(0, _CH)],
                                  osm[b]).wait()

        def add(b):
            def row(r, c2):
                for c in range(_D // _LANES):
                    sl = pl.ds(c * _LANES, _LANES)
                    ob[b, r, sl] = tkb[b, r, sl] + psb[b, r, sl]
                return c2

            lax.fori_loop(0, _CH, row, 0)

        for b in range(_NBUF):
            start_g(b, b)
        for b in range(_NBUF):
            wait_g(b)
            add(b)
            start_o(b, b)
            start_g(b + _NBUF, b)

        def pair(g, carry):
            for b in range(_NBUF):
                i = g * _NBUF + b
                wait_g(b)
                wait_o(b)
                add(b)
                start_o(i, b)
                start_g(i + _NBUF, b)
            return carry

        lax.fori_loop(1, _NCH // _NBUF - 1, pair, 0)

        for b in range(_NBUF):
            i = _NCH - _NBUF + b
            wait_g(b)
            wait_o(b)
            add(b)
            start_o(i, b)
        for b in range(_NBUF):
            wait_o(b)

    return body


_kernel_fn = _make_kernel()


def kernel(input_ids, token_table, pos_table):
    ids = input_ids.astype(jnp.int32).reshape(_NW, _NCH, _CH)
    out = _kernel_fn(ids, token_table, pos_table)
    return out.reshape(_B, _S, _D)
